# baseline (speedup 1.0000x reference)
import functools

import jax
import jax.numpy as jnp
from jax import lax
from jax.experimental import pallas as pl
from jax.experimental.pallas import tpu as pltpu
from jax.experimental.pallas import tpu_sc as plsc

N = 10000
E = 320000
F_IN = 128
H = 8
C = 32
HC = H * C
G = 64
NCLS = 10

SC_CORES = 2
SC_TILES = 16
LANES = 16
HPC = H // SC_CORES
FPC = HPC * C
WROW = FPC + LANES

E_PER_TILE = E // SC_TILES
EB = 80
N_CHUNKS = E_PER_TILE // EB
SUP = 10
NSUP = N_CHUNKS // SUP
NPT_STEP = 624
NPT_LEN = 640

BN = 2000
NBLK = N // BN


def _dense1_body(x_ref, w_ref, a_ref, zs_ref, asad_ref):
    z = jnp.dot(x_ref[...], w_ref[...], preferred_element_type=jnp.float32)
    asad = jnp.dot(z, a_ref[...], preferred_element_type=jnp.float32)
    zb = z.astype(jnp.bfloat16)
    zs_ref[0, :, :] = zb[:, :FPC]
    zs_ref[1, :, :] = zb[:, FPC:]
    asad_ref[0, :, :] = asad
    asad_ref[1, :, :] = asad


_dense1 = pl.pallas_call(
    _dense1_body,
    grid=(NBLK,),
    in_specs=[
        pl.BlockSpec((BN, F_IN), lambda i: (i, 0)),
        pl.BlockSpec((F_IN, HC), lambda i: (0, 0)),
        pl.BlockSpec((HC, 2 * H), lambda i: (0, 0)),
    ],
    out_specs=[
        pl.BlockSpec((SC_CORES, BN, FPC), lambda i: (0, i, 0)),
        pl.BlockSpec((SC_CORES, BN, 2 * H), lambda i: (0, i, 0)),
    ],
    out_shape=[
        jax.ShapeDtypeStruct((SC_CORES, N, FPC), jnp.bfloat16),
        jax.ShapeDtypeStruct((SC_CORES, N, 2 * H), jnp.float32),
    ],
)


def _normalize(a0_ref, a1_ref, rep0_ref, rep1_ref, b_ref):
    denx = (jnp.dot(1.0 / (a0_ref[0, :, FPC:FPC + HPC] + 1e-16), rep0_ref[...],
                    preferred_element_type=jnp.float32)
            + jnp.dot(1.0 / (a1_ref[0, :, FPC:FPC + HPC] + 1e-16), rep1_ref[...],
                      preferred_element_type=jnp.float32))
    acc = jnp.concatenate([a0_ref[0, :, :FPC], a1_ref[0, :, :FPC]], axis=1)
    return jnp.maximum(acc * denx + b_ref[...], 0.0)


def _dense2_body(a0_ref, a1_ref, w_ref, a2_ref, b_ref,
                 rep0_ref, rep1_ref, zs_ref, asad_ref):
    h = _normalize(a0_ref, a1_ref, rep0_ref, rep1_ref, b_ref)
    z = jnp.dot(h, w_ref[...], preferred_element_type=jnp.float32)
    asad = jnp.dot(z, a2_ref[...], preferred_element_type=jnp.float32)
    zb = z.astype(jnp.bfloat16)
    zs_ref[0, :, :] = zb[:, :FPC]
    zs_ref[1, :, :] = zb[:, FPC:]
    asad_ref[0, :, :] = asad
    asad_ref[1, :, :] = asad


_dense2 = pl.pallas_call(
    _dense2_body,
    grid=(NBLK,),
    in_specs=[
        pl.BlockSpec((1, BN, WROW), lambda i: (0, i, 0)),
        pl.BlockSpec((1, BN, WROW), lambda i: (1, i, 0)),
        pl.BlockSpec((HC, HC), lambda i: (0, 0)),
        pl.BlockSpec((HC, 2 * H), lambda i: (0, 0)),
        pl.BlockSpec((1, HC), lambda i: (0, 0)),
        pl.BlockSpec((HPC, HC), lambda i: (0, 0)),
        pl.BlockSpec((HPC, HC), lambda i: (0, 0)),
    ],
    out_specs=[
        pl.BlockSpec((SC_CORES, BN, FPC), lambda i: (0, i, 0)),
        pl.BlockSpec((SC_CORES, BN, 2 * H), lambda i: (0, i, 0)),
    ],
    out_shape=[
        jax.ShapeDtypeStruct((SC_CORES, N, FPC), jnp.bfloat16),
        jax.ShapeDtypeStruct((SC_CORES, N, 2 * H), jnp.float32),
    ],
)


def _final_body(a0_ref, a1_ref, bat_ref, b_ref, rep0_ref, rep1_ref,
                wfc_ref, bfc_ref, out_ref, pooled_ref, cnt_ref):
    i = pl.program_id(0)

    @pl.when(i == 0)
    def _():
        pooled_ref[...] = jnp.zeros((G, HC), jnp.float32)
        cnt_ref[...] = jnp.zeros((G, 1), jnp.float32)

    h = _normalize(a0_ref, a1_ref, rep0_ref, rep1_ref, b_ref)

    gids = lax.broadcasted_iota(jnp.int32, (G, BN), 0)
    onehot = jnp.where(gids == bat_ref[0, :, :], 1.0, 0.0)
    pooled_ref[...] += jnp.dot(onehot, h, preferred_element_type=jnp.float32)
    cnt_ref[...] += jnp.sum(onehot, axis=1, keepdims=True)

    pooled = pooled_ref[...] / jnp.maximum(cnt_ref[...], 1.0)
    out_ref[...] = (jnp.dot(pooled, wfc_ref[...],
                            preferred_element_type=jnp.float32) + bfc_ref[...])


_final = pl.pallas_call(
    _final_body,
    grid=(NBLK,),
    in_specs=[
        pl.BlockSpec((1, BN, WROW), lambda i: (0, i, 0)),
        pl.BlockSpec((1, BN, WROW), lambda i: (1, i, 0)),
        pl.BlockSpec((1, 1, BN), lambda i: (i, 0, 0)),
        pl.BlockSpec((1, HC), lambda i: (0, 0)),
        pl.BlockSpec((HPC, HC), lambda i: (0, 0)),
        pl.BlockSpec((HPC, HC), lambda i: (0, 0)),
        pl.BlockSpec((HC, NCLS), lambda i: (0, 0)),
        pl.BlockSpec((1, NCLS), lambda i: (0, 0)),
    ],
    out_specs=pl.BlockSpec((G, NCLS), lambda i: (0, 0)),
    out_shape=jax.ShapeDtypeStruct((G, NCLS), jnp.float32),
    scratch_shapes=[
        pltpu.VMEM((G, HC), jnp.float32),
        pltpu.VMEM((G, 1), jnp.float32),
    ],
)


@functools.cache
def _make_edge_kernel():
    sc_mesh = plsc.VectorSubcoreMesh(
        core_axis_name="c", subcore_axis_name="s",
        num_cores=SC_CORES, num_subcores=SC_TILES)
    return pl.kernel(
        _edge_body,
        out_type=jax.ShapeDtypeStruct((SC_CORES * N, WROW), jnp.float32),
        mesh=sc_mesh,
        compiler_params=pltpu.CompilerParams(
            needs_layout_passes=False, use_tc_tiling_on_sc=False),
        scratch_types=[
            pltpu.VMEM((SUP * EB,), jnp.int32),
            pltpu.VMEM((SUP * EB,), jnp.int32),
            pltpu.VMEM((SUP, EB), jnp.int32),
            pltpu.VMEM((SUP, EB), jnp.int32),
            pltpu.VMEM((EB, FPC), jnp.bfloat16),
            pltpu.VMEM((EB, FPC), jnp.bfloat16),
            pltpu.VMEM((EB, LANES), jnp.float32),
            pltpu.VMEM((EB, LANES), jnp.float32),
            pltpu.VMEM((EB, LANES), jnp.float32),
            pltpu.VMEM((EB, LANES), jnp.float32),
            pltpu.VMEM((EB, WROW), jnp.float32),
            pltpu.VMEM_SHARED((N, WROW), jnp.float32),
            pltpu.SemaphoreType.DMA,
            pltpu.SemaphoreType.DMA,
            pltpu.SemaphoreType.DMA,
        ],
    )


def _edge_body(zbf, asadd, src, dst, accden_out,
               sstage, dstage, zidx2d, didx2d, zrb0, zrb1,
               ars0, ars1, ard0, ard1, msg, accsh, semg0, semg1, sems):
    cid = lax.axis_index("c")
    tid = lax.axis_index("s")
    lane = lax.iota(jnp.int32, LANES)
    zero16 = jnp.zeros((LANES,), jnp.float32)
    bufs = ((zrb0, ars0, ard0, semg0),
            (zrb1, ars1, ard1, semg1))

    def _zero_buf(r, carry):
        for q in range(WROW // LANES):
            msg[r, pl.ds(q * LANES, LANES)] = zero16
        return carry
    lax.fori_loop(0, EB, _zero_buf, 0)

    row0 = tid * NPT_STEP
    for p in range(NPT_LEN // EB):
        pltpu.sync_copy(msg, accsh.at[pl.ds(row0 + p * EB, EB)])
    plsc.subcore_barrier()

    ebase = tid * E_PER_TILE
    coff = cid * N
    hbase = cid * HPC

    def prefetch(j, p):
        zrb, ars, ard, semg = bufs[p]
        pltpu.async_copy(zbf.at[zidx2d.at[j]], zrb, semg)
        pltpu.async_copy(asadd.at[zidx2d.at[j]], ars, semg)
        pltpu.async_copy(asadd.at[didx2d.at[j]], ard, semg)

    def wait_gathers(j, p):
        zrb, ars, ard, semg = bufs[p]
        pltpu.make_async_copy(zbf.at[zidx2d.at[j]], zrb, semg).wait()
        pltpu.make_async_copy(asadd.at[zidx2d.at[j]], ars, semg).wait()
        pltpu.make_async_copy(asadd.at[didx2d.at[j]], ard, semg).wait()

    def drain_scatter(j):
        pltpu.make_async_copy(msg, accsh.at[didx2d.at[j]], sems).wait()

    def process(j, p, g):
        zrb, ars, ard, semg = bufs[p]

        if j >= 1:
            drain_scatter(j - 1)
        else:
            @pl.when(g >= 1)
            def _():
                drain_scatter(SUP - 1)

        @plsc.parallel_loop(0, EB // LANES)
        def _wgrp(jj):
            eidx = jj * LANES + lane
            for hh in range(HPC):
                ca = jnp.zeros((LANES,), jnp.int32) + (hbase + hh)
                cd = jnp.zeros((LANES,), jnp.int32) + (H + hbase + hh)
                av = plsc.load_gather(ars, [eidx, ca])
                bv = plsc.load_gather(ard, [eidx, cd])
                t = av + bv
                t = jnp.where(t > 0, t, 0.2 * t)
                w = jnp.exp(t)
                plsc.store_scatter(
                    msg, [eidx, jnp.zeros((LANES,), jnp.int32) + (FPC + hh)], w)

        @plsc.parallel_loop(0, EB, unroll=2)
        def _scale(e):
            wrow = msg[e, pl.ds(FPC, LANES)]
            for hh in range(HPC):
                w = wrow[hh]
                v32 = zrb[e, pl.ds(hh * C, C)]
                va, vb = plsc.unpack(v32, format=plsc.PackFormat.INTERLEAVED)
                msg[e, pl.ds(hh * C, LANES)] = va * w
                msg[e, pl.ds(hh * C + LANES, LANES)] = vb * w

        pltpu.async_copy(msg, accsh.at[didx2d.at[j]], sems, add=True)

    def g_body(g, carry):
        base = ebase + g * (SUP * EB)
        pltpu.sync_copy(src.at[pl.ds(base, SUP * EB)], sstage)
        pltpu.sync_copy(dst.at[pl.ds(base, SUP * EB)], dstage)

        @plsc.parallel_loop(0, SUP)
        def _tr(q):
            for v in range(EB // LANES):
                sl = pl.ds(v * LANES, LANES)
                zidx2d[q, sl] = sstage[pl.ds(q * EB + v * LANES, LANES)] + coff
                didx2d[q, sl] = dstage[pl.ds(q * EB + v * LANES, LANES)]

        prefetch(0, 0)
        for j in range(SUP):
            p = j & 1
            wait_gathers(j, p)
            if j + 1 < SUP:
                prefetch(j + 1, 1 - p)
            process(j, p, g)
        return carry

    lax.fori_loop(0, NSUP, g_body, 0)
    drain_scatter(SUP - 1)
    plsc.subcore_barrier()

    orow = coff + row0
    pltpu.sync_copy(accsh.at[pl.ds(row0, NPT_LEN)],
                    accden_out.at[pl.ds(orow, NPT_LEN)])


def _build_A(asrc, adst):
    eye = jnp.eye(H, dtype=jnp.float32)
    As = (asrc[:, :, None] * eye[:, None, :]).reshape(HC, H)
    Ad = (adst[:, :, None] * eye[:, None, :]).reshape(HC, H)
    return jnp.concatenate([As, Ad], axis=1)


def _build_reps():
    rep_full = (jnp.eye(H, dtype=jnp.float32)[:, :, None]
                * jnp.ones((1, 1, C), jnp.float32)).reshape(H, HC)
    return rep_full[:HPC], rep_full[HPC:]


def _build_perm():
    chan = jnp.arange(HC)
    head = chan // C
    t = chan % C
    mempos = jnp.where(t < C // 2,
                       head * C + 2 * t,
                       head * C + 2 * (t - C // 2) + 1)
    return jnp.zeros((HC, HC), jnp.float32).at[chan, mempos].set(1.0)


def kernel(x, edge_index, batch, W1, a1_src, a1_dst, b1,
           W2, a2_src, a2_dst, b2, Wfc, bfc):
    src = edge_index[0]
    dst = edge_index[1]
    P = _build_perm()
    A1 = _build_A(a1_src, a1_dst)
    A2 = _build_A(a2_src, a2_dst)
    rep0, rep1 = _build_reps()
    W1p = W1 @ P
    A1p = P.T @ A1
    W2p = W2 @ P
    A2p = P.T @ A2

    zs1, asad1 = _dense1(x, W1p, A1p)
    acc1 = _make_edge_kernel()(
        zs1.reshape(SC_CORES * N, FPC), asad1.reshape(SC_CORES * N, 2 * H),
        src, dst)
    acc1 = acc1.reshape(SC_CORES, N, WROW)

    zs2, asad2 = _dense2(acc1, acc1, W2p, A2p, b1.reshape(1, HC), rep0, rep1)
    acc2 = _make_edge_kernel()(
        zs2.reshape(SC_CORES * N, FPC), asad2.reshape(SC_CORES * N, 2 * H),
        src, dst)
    acc2 = acc2.reshape(SC_CORES, N, WROW)

    out = _final(acc2, acc2, batch.reshape(NBLK, 1, BN), b2.reshape(1, HC),
                 rep0, rep1, Wfc, bfc.reshape(1, NCLS))
    return out


# speedup vs baseline: 1.0294x; 1.0294x over previous
import functools

import jax
import jax.numpy as jnp
from jax import lax
from jax.experimental import pallas as pl
from jax.experimental.pallas import tpu as pltpu
from jax.experimental.pallas import tpu_sc as plsc

N = 10000
E = 320000
F_IN = 128
H = 8
C = 32
HC = H * C
G = 64
NCLS = 10

SC_CORES = 2
SC_TILES = 16
LANES = 16
HPC = H // SC_CORES
FPC = HPC * C
WROW = FPC + LANES

E_PER_TILE = E // SC_TILES
EB = 80
N_CHUNKS = E_PER_TILE // EB
SUP = 10
NSUP = N_CHUNKS // SUP
NPT_STEP = 624
NPT_LEN = 640

BN = 2000
NBLK = N // BN


def _dense1_body(x_ref, w_ref, a_ref, zs_ref, asad_ref):
    z = jnp.dot(x_ref[...], w_ref[...], preferred_element_type=jnp.float32)
    asad = jnp.dot(z, a_ref[...], preferred_element_type=jnp.float32)
    zb = z.astype(jnp.bfloat16)
    zs_ref[0, :, :] = zb[:, :FPC]
    zs_ref[1, :, :] = zb[:, FPC:]
    asad_ref[0, :, :] = asad
    asad_ref[1, :, :] = asad


_dense1 = pl.pallas_call(
    _dense1_body,
    grid=(NBLK,),
    in_specs=[
        pl.BlockSpec((BN, F_IN), lambda i: (i, 0)),
        pl.BlockSpec((F_IN, HC), lambda i: (0, 0)),
        pl.BlockSpec((HC, 2 * H), lambda i: (0, 0)),
    ],
    out_specs=[
        pl.BlockSpec((SC_CORES, BN, FPC), lambda i: (0, i, 0)),
        pl.BlockSpec((SC_CORES, BN, 2 * H), lambda i: (0, i, 0)),
    ],
    out_shape=[
        jax.ShapeDtypeStruct((SC_CORES, N, FPC), jnp.bfloat16),
        jax.ShapeDtypeStruct((SC_CORES, N, 2 * H), jnp.float32),
    ],
)


def _normalize(a0_ref, a1_ref, d0_ref, d1_ref, rep0_ref, rep1_ref, b_ref):
    denx = (jnp.dot(1.0 / (d0_ref[0, :, :HPC] + 1e-16), rep0_ref[...],
                    preferred_element_type=jnp.float32)
            + jnp.dot(1.0 / (d1_ref[0, :, :HPC] + 1e-16), rep1_ref[...],
                      preferred_element_type=jnp.float32))
    acc = jnp.concatenate([a0_ref[0, :, :], a1_ref[0, :, :]], axis=1)
    return jnp.maximum(acc * denx + b_ref[...], 0.0)


def _dense2_body(a0_ref, a1_ref, d0_ref, d1_ref, w_ref, a2_ref, b_ref,
                 rep0_ref, rep1_ref, zs_ref, asad_ref):
    h = _normalize(a0_ref, a1_ref, d0_ref, d1_ref, rep0_ref, rep1_ref, b_ref)
    z = jnp.dot(h, w_ref[...], preferred_element_type=jnp.float32)
    asad = jnp.dot(z, a2_ref[...], preferred_element_type=jnp.float32)
    zb = z.astype(jnp.bfloat16)
    zs_ref[0, :, :] = zb[:, :FPC]
    zs_ref[1, :, :] = zb[:, FPC:]
    asad_ref[0, :, :] = asad
    asad_ref[1, :, :] = asad


_dense2 = pl.pallas_call(
    _dense2_body,
    grid=(NBLK,),
    in_specs=[
        pl.BlockSpec((1, BN, FPC), lambda i: (0, i, 0)),
        pl.BlockSpec((1, BN, FPC), lambda i: (1, i, 0)),
        pl.BlockSpec((1, BN, LANES), lambda i: (0, i, 0)),
        pl.BlockSpec((1, BN, LANES), lambda i: (1, i, 0)),
        pl.BlockSpec((HC, HC), lambda i: (0, 0)),
        pl.BlockSpec((HC, 2 * H), lambda i: (0, 0)),
        pl.BlockSpec((1, HC), lambda i: (0, 0)),
        pl.BlockSpec((HPC, HC), lambda i: (0, 0)),
        pl.BlockSpec((HPC, HC), lambda i: (0, 0)),
    ],
    out_specs=[
        pl.BlockSpec((SC_CORES, BN, FPC), lambda i: (0, i, 0)),
        pl.BlockSpec((SC_CORES, BN, 2 * H), lambda i: (0, i, 0)),
    ],
    out_shape=[
        jax.ShapeDtypeStruct((SC_CORES, N, FPC), jnp.bfloat16),
        jax.ShapeDtypeStruct((SC_CORES, N, 2 * H), jnp.float32),
    ],
)


def _final_body(a0_ref, a1_ref, d0_ref, d1_ref, bat_ref, b_ref,
                rep0_ref, rep1_ref,
                wfc_ref, bfc_ref, out_ref, pooled_ref, cnt_ref):
    i = pl.program_id(0)

    @pl.when(i == 0)
    def _():
        pooled_ref[...] = jnp.zeros((G, HC), jnp.float32)
        cnt_ref[...] = jnp.zeros((G, 1), jnp.float32)

    h = _normalize(a0_ref, a1_ref, d0_ref, d1_ref, rep0_ref, rep1_ref, b_ref)

    gids = lax.broadcasted_iota(jnp.int32, (G, BN), 0)
    onehot = jnp.where(gids == bat_ref[0, :, :], 1.0, 0.0)
    pooled_ref[...] += jnp.dot(onehot, h, preferred_element_type=jnp.float32)
    cnt_ref[...] += jnp.sum(onehot, axis=1, keepdims=True)

    pooled = pooled_ref[...] / jnp.maximum(cnt_ref[...], 1.0)
    out_ref[...] = (jnp.dot(pooled, wfc_ref[...],
                            preferred_element_type=jnp.float32) + bfc_ref[...])


_final = pl.pallas_call(
    _final_body,
    grid=(NBLK,),
    in_specs=[
        pl.BlockSpec((1, BN, FPC), lambda i: (0, i, 0)),
        pl.BlockSpec((1, BN, FPC), lambda i: (1, i, 0)),
        pl.BlockSpec((1, BN, LANES), lambda i: (0, i, 0)),
        pl.BlockSpec((1, BN, LANES), lambda i: (1, i, 0)),
        pl.BlockSpec((1, 1, BN), lambda i: (i, 0, 0)),
        pl.BlockSpec((1, HC), lambda i: (0, 0)),
        pl.BlockSpec((HPC, HC), lambda i: (0, 0)),
        pl.BlockSpec((HPC, HC), lambda i: (0, 0)),
        pl.BlockSpec((HC, NCLS), lambda i: (0, 0)),
        pl.BlockSpec((1, NCLS), lambda i: (0, 0)),
    ],
    out_specs=pl.BlockSpec((G, NCLS), lambda i: (0, 0)),
    out_shape=jax.ShapeDtypeStruct((G, NCLS), jnp.float32),
    scratch_shapes=[
        pltpu.VMEM((G, HC), jnp.float32),
        pltpu.VMEM((G, 1), jnp.float32),
    ],
)


@functools.cache
def _make_edge_kernel():
    sc_mesh = plsc.VectorSubcoreMesh(
        core_axis_name="c", subcore_axis_name="s",
        num_cores=SC_CORES, num_subcores=SC_TILES)
    return pl.kernel(
        _edge_body,
        out_type=[
            jax.ShapeDtypeStruct((SC_CORES * N, FPC), jnp.float32),
            jax.ShapeDtypeStruct((SC_CORES * N, LANES), jnp.float32),
        ],
        mesh=sc_mesh,
        compiler_params=pltpu.CompilerParams(
            needs_layout_passes=False, use_tc_tiling_on_sc=False),
        scratch_types=[
            pltpu.VMEM((SUP * EB,), jnp.int32),
            pltpu.VMEM((SUP * EB,), jnp.int32),
            pltpu.VMEM((SUP, EB), jnp.int32),
            pltpu.VMEM((SUP, EB), jnp.int32),
            pltpu.VMEM((EB, FPC), jnp.bfloat16),
            pltpu.VMEM((EB, FPC), jnp.bfloat16),
            pltpu.VMEM((EB, LANES), jnp.float32),
            pltpu.VMEM((EB, LANES), jnp.float32),
            pltpu.VMEM((EB, LANES), jnp.float32),
            pltpu.VMEM((EB, LANES), jnp.float32),
            pltpu.VMEM((EB, WROW), jnp.float32),
            pltpu.VMEM_SHARED((N, WROW), jnp.float32),
            pltpu.SemaphoreType.DMA,
            pltpu.SemaphoreType.DMA,
            pltpu.SemaphoreType.DMA,
        ],
    )


def _edge_body(zbf, asadd, src, dst, acc_out, den_out,
               sstage, dstage, zidx2d, didx2d, zrb0, zrb1,
               ars0, ars1, ard0, ard1, msg, accsh, semg0, semg1, sems):
    cid = lax.axis_index("c")
    tid = lax.axis_index("s")
    lane = lax.iota(jnp.int32, LANES)
    zero16 = jnp.zeros((LANES,), jnp.float32)
    bufs = ((zrb0, ars0, ard0, semg0),
            (zrb1, ars1, ard1, semg1))

    def _zero_buf(r, carry):
        for q in range(WROW // LANES):
            msg[r, pl.ds(q * LANES, LANES)] = zero16
        return carry
    lax.fori_loop(0, EB, _zero_buf, 0)

    row0 = tid * NPT_STEP
    for p in range(NPT_LEN // EB):
        pltpu.sync_copy(msg, accsh.at[pl.ds(row0 + p * EB, EB)])
    plsc.subcore_barrier()

    ebase = tid * E_PER_TILE
    coff = cid * N
    hbase = cid * HPC

    def prefetch(j, p):
        zrb, ars, ard, semg = bufs[p]
        pltpu.async_copy(zbf.at[zidx2d.at[j]], zrb, semg)
        pltpu.async_copy(asadd.at[zidx2d.at[j]], ars, semg)
        pltpu.async_copy(asadd.at[didx2d.at[j]], ard, semg)

    def wait_gathers(j, p):
        zrb, ars, ard, semg = bufs[p]
        pltpu.make_async_copy(zbf.at[zidx2d.at[j]], zrb, semg).wait()
        pltpu.make_async_copy(asadd.at[zidx2d.at[j]], ars, semg).wait()
        pltpu.make_async_copy(asadd.at[didx2d.at[j]], ard, semg).wait()

    def drain_scatter(j):
        pltpu.make_async_copy(msg, accsh.at[didx2d.at[j]], sems).wait()

    def process(j, p, g):
        zrb, ars, ard, semg = bufs[p]

        if j >= 1:
            drain_scatter(j - 1)
        else:
            @pl.when(g >= 1)
            def _():
                drain_scatter(SUP - 1)

        @plsc.parallel_loop(0, EB // LANES)
        def _wgrp(jj):
            eidx = jj * LANES + lane
            for hh in range(HPC):
                ca = jnp.zeros((LANES,), jnp.int32) + (hbase + hh)
                cd = jnp.zeros((LANES,), jnp.int32) + (H + hbase + hh)
                av = plsc.load_gather(ars, [eidx, ca])
                bv = plsc.load_gather(ard, [eidx, cd])
                t = av + bv
                t = jnp.where(t > 0, t, 0.2 * t)
                w = jnp.exp(t)
                plsc.store_scatter(
                    msg, [eidx, jnp.zeros((LANES,), jnp.int32) + (FPC + hh)], w)

        @plsc.parallel_loop(0, EB, unroll=2)
        def _scale(e):
            wrow = msg[e, pl.ds(FPC, LANES)]
            for hh in range(HPC):
                w = wrow[hh]
                v32 = zrb[e, pl.ds(hh * C, C)]
                va, vb = plsc.unpack(v32, format=plsc.PackFormat.INTERLEAVED)
                msg[e, pl.ds(hh * C, LANES)] = va * w
                msg[e, pl.ds(hh * C + LANES, LANES)] = vb * w

        pltpu.async_copy(msg, accsh.at[didx2d.at[j]], sems, add=True)

    def g_body(g, carry):
        base = ebase + g * (SUP * EB)
        pltpu.sync_copy(src.at[pl.ds(base, SUP * EB)], sstage)
        pltpu.sync_copy(dst.at[pl.ds(base, SUP * EB)], dstage)

        @plsc.parallel_loop(0, SUP)
        def _tr(q):
            for v in range(EB // LANES):
                sl = pl.ds(v * LANES, LANES)
                zidx2d[q, sl] = sstage[pl.ds(q * EB + v * LANES, LANES)] + coff
                didx2d[q, sl] = dstage[pl.ds(q * EB + v * LANES, LANES)]

        prefetch(0, 0)
        for j in range(SUP):
            p = j & 1
            wait_gathers(j, p)
            if j + 1 < SUP:
                prefetch(j + 1, 1 - p)
            process(j, p, g)
        return carry

    lax.fori_loop(0, NSUP, g_body, 0)
    drain_scatter(SUP - 1)
    plsc.subcore_barrier()

    orow = coff + row0
    pltpu.sync_copy(accsh.at[pl.ds(row0, NPT_LEN), pl.ds(0, FPC)],
                    acc_out.at[pl.ds(orow, NPT_LEN)])
    pltpu.sync_copy(accsh.at[pl.ds(row0, NPT_LEN), pl.ds(FPC, LANES)],
                    den_out.at[pl.ds(orow, NPT_LEN)])


def _build_A(asrc, adst):
    eye = jnp.eye(H, dtype=jnp.float32)
    As = (asrc[:, :, None] * eye[:, None, :]).reshape(HC, H)
    Ad = (adst[:, :, None] * eye[:, None, :]).reshape(HC, H)
    return jnp.concatenate([As, Ad], axis=1)


def _build_reps():
    rep_full = (jnp.eye(H, dtype=jnp.float32)[:, :, None]
                * jnp.ones((1, 1, C), jnp.float32)).reshape(H, HC)
    return rep_full[:HPC], rep_full[HPC:]


def _build_perm():
    chan = jnp.arange(HC)
    head = chan // C
    t = chan % C
    mempos = jnp.where(t < C // 2,
                       head * C + 2 * t,
                       head * C + 2 * (t - C // 2) + 1)
    return jnp.zeros((HC, HC), jnp.float32).at[chan, mempos].set(1.0)


def kernel(x, edge_index, batch, W1, a1_src, a1_dst, b1,
           W2, a2_src, a2_dst, b2, Wfc, bfc):
    src = edge_index[0]
    dst = edge_index[1]
    P = _build_perm()
    A1 = _build_A(a1_src, a1_dst)
    A2 = _build_A(a2_src, a2_dst)
    rep0, rep1 = _build_reps()
    W1p = W1 @ P
    A1p = P.T @ A1
    W2p = W2 @ P
    A2p = P.T @ A2

    zs1, asad1 = _dense1(x, W1p, A1p)
    acc1, den1 = _make_edge_kernel()(
        zs1.reshape(SC_CORES * N, FPC), asad1.reshape(SC_CORES * N, 2 * H),
        src, dst)
    acc1 = acc1.reshape(SC_CORES, N, FPC)
    den1 = den1.reshape(SC_CORES, N, LANES)

    zs2, asad2 = _dense2(acc1, acc1, den1, den1, W2p, A2p,
                         b1.reshape(1, HC), rep0, rep1)
    acc2, den2 = _make_edge_kernel()(
        zs2.reshape(SC_CORES * N, FPC), asad2.reshape(SC_CORES * N, 2 * H),
        src, dst)
    acc2 = acc2.reshape(SC_CORES, N, FPC)
    den2 = den2.reshape(SC_CORES, N, LANES)

    out = _final(acc2, acc2, den2, den2, batch.reshape(NBLK, 1, BN),
                 b2.reshape(1, HC), rep0, rep1, Wfc, bfc.reshape(1, NCLS))
    return out
